# Initial kernel scaffold; baseline (speedup 1.0000x reference)
import jax
import jax.numpy as jnp
from jax.experimental import pallas as pl


def kernel(pos_head, pos_rel, pos_tail, neg_head, neg_rel, neg_tail, entity_table, relation_table, normal_table):
    raise NotImplementedError("write your pallas kernel here")


# speedup vs baseline: 1.9122x; 1.9122x over previous
import functools

import jax
import jax.numpy as jnp
from jax import lax
from jax.experimental import pallas as pl
from jax.experimental.pallas import tpu as pltpu
from jax.experimental.pallas import tpu_sc as plsc

B = 128
D = 128
NS = 16
RPW = B // NS
NCH = D // 16
MARGIN_ = 1.0

_mesh = plsc.VectorSubcoreMesh(core_axis_name="c", subcore_axis_name="s")


@functools.partial(
    pl.kernel,
    out_type=(jax.ShapeDtypeStruct((16,), jnp.float32),
              jax.ShapeDtypeStruct((NS, 2, 16), jnp.float32)),
    mesh=_mesh,
    compiler_params=pltpu.CompilerParams(needs_layout_passes=False),
    scratch_types=[
        pltpu.VMEM((RPW,), jnp.int32),
        pltpu.VMEM((RPW,), jnp.int32),
        pltpu.VMEM((RPW,), jnp.int32),
        pltpu.VMEM((RPW,), jnp.int32),
        pltpu.VMEM((RPW,), jnp.int32),
        pltpu.VMEM((B,), jnp.int32),
        pltpu.VMEM((B,), jnp.int32),
        pltpu.VMEM((RPW, D), jnp.float32),
        pltpu.VMEM((RPW, D), jnp.float32),
        pltpu.VMEM((RPW, D), jnp.float32),
        pltpu.VMEM((RPW, D), jnp.float32),
        pltpu.VMEM((RPW, D), jnp.float32),
        pltpu.VMEM((B, D), jnp.float32),
        pltpu.VMEM((B, D), jnp.float32),
        pltpu.VMEM((2, 16), jnp.float32),
        pltpu.VMEM((NS, 2, 16), jnp.float32),
        pltpu.VMEM((16,), jnp.float32),
        pltpu.SemaphoreType.DMA,
    ],
)
def _transh_sc(ph_hbm, prl_hbm, pt_hbm, nrl_hbm, nt_hbm,
               ent_tab, rel_tab, nrm_tab, out_hbm, stage_hbm,
               idx_ph, idx_pt, idx_nt, idx_pr, idx_nr, idx_prf, idx_nrf,
               eh, et, ent, pr, nr, pn, nn,
               part_v, all_v, out_v, sem):
    c = lax.axis_index("c")
    s = lax.axis_index("s")
    base = s * RPW

    pltpu.sync_copy(ph_hbm.at[pl.ds(base, RPW)], idx_ph)
    pltpu.sync_copy(pt_hbm.at[pl.ds(base, RPW)], idx_pt)
    pltpu.sync_copy(nt_hbm.at[pl.ds(base, RPW)], idx_nt)
    pltpu.sync_copy(prl_hbm.at[pl.ds(base, RPW)], idx_pr)
    pltpu.sync_copy(nrl_hbm.at[pl.ds(base, RPW)], idx_nr)
    pltpu.sync_copy(prl_hbm, idx_prf)
    pltpu.sync_copy(nrl_hbm, idx_nrf)

    cps = [
        pltpu.async_copy(ent_tab.at[idx_ph], eh, sem),
        pltpu.async_copy(ent_tab.at[idx_pt], et, sem),
        pltpu.async_copy(ent_tab.at[idx_nt], ent, sem),
        pltpu.async_copy(rel_tab.at[idx_pr], pr, sem),
        pltpu.async_copy(rel_tab.at[idx_nr], nr, sem),
        pltpu.async_copy(nrm_tab.at[idx_prf], pn, sem),
        pltpu.async_copy(nrm_tab.at[idx_nrf], nn, sem),
    ]
    for cp in cps:
        cp.wait()

    iota = lax.iota(jnp.int32, 16)

    def body(k, carry):
        accp, accn = carry
        i_loc = k // NCH
        ch = k % NCH
        col = ch * 16
        i_glob = base + i_loc
        j_vec = col + iota
        i_vec = jnp.full((16,), i_glob, jnp.int32)
        eh_v = eh[i_loc, pl.ds(col, 16)]
        et_v = et[i_loc, pl.ds(col, 16)]
        ent_v = ent[i_loc, pl.ds(col, 16)]
        pr_v = pr[i_loc, pl.ds(col, 16)]
        nr_v = nr[i_loc, pl.ds(col, 16)]
        pn_row = pn[i_glob, pl.ds(col, 16)]
        nn_row = nn[i_glob, pl.ds(col, 16)]
        pnT = plsc.load_gather(pn, [j_vec, i_vec])
        nnT = plsc.load_gather(nn, [j_vec, i_vec])
        fp = 1.0 - pnT * pn_row
        fn = 1.0 - nnT * nn_row
        a_pos = (eh_v - et_v) * fp + pr_v
        a_neg = eh_v * fp - ent_v * fn + nr_v
        return accp + jnp.abs(a_pos), accn + jnp.abs(a_neg)

    zero = jnp.zeros((16,), jnp.float32)
    accp, accn = lax.fori_loop(0, RPW * NCH, body, (zero, zero))

    part_v[0, :] = accp
    part_v[1, :] = accn

    @pl.when(c == 0)
    def _():
        pltpu.sync_copy(part_v, stage_hbm.at[s])

    plsc.subcore_barrier()

    @pl.when(jnp.logical_and(c == 0, s == 0))
    def _():
        pltpu.sync_copy(stage_hbm, all_v)
        sp = zero
        sn = zero
        for w in range(NS):
            sp = sp + all_v[w, 0, :]
            sn = sn + all_v[w, 1, :]
        s_pos = jnp.sum(sp)
        s_neg = jnp.sum(sn)
        loss = jnp.maximum(0.0, s_neg * s_neg - s_pos * s_pos + MARGIN_)
        out_v[...] = jnp.full((16,), loss, jnp.float32)
        pltpu.sync_copy(out_v, out_hbm)


def kernel(pos_head, pos_rel, pos_tail, neg_head, neg_rel, neg_tail,
           entity_table, relation_table, normal_table):
    del neg_head
    out, _ = _transh_sc(
        pos_head.astype(jnp.int32),
        pos_rel.astype(jnp.int32),
        pos_tail.astype(jnp.int32),
        neg_rel.astype(jnp.int32),
        neg_tail.astype(jnp.int32),
        entity_table,
        relation_table,
        normal_table,
    )
    return out[0]
